# baseline (speedup 1.0000x reference)
import functools

import jax
import jax.numpy as jnp
from jax import lax
from jax.experimental import pallas as pl
from jax.experimental.pallas import tpu as pltpu
from jax.experimental.pallas import tpu_sc as plsc

BS = 512


def _ln_kernel(x_ref, p_ref, w_ref, b_ref, o_ref):
    e = x_ref[...] + p_ref[None]
    mean = jnp.mean(e, axis=-1, keepdims=True)
    c = e - mean
    var = jnp.mean(c * c, axis=-1, keepdims=True)
    inv = jax.lax.rsqrt(var + 1e-5)
    o_ref[...] = (c * inv) * w_ref[0] + b_ref[0]


@jax.jit
def _run_tc(x, pos_table, ln_w, ln_b):
    B, S, D = x.shape
    grid = (S // BS,)
    return pl.pallas_call(
        _ln_kernel,
        grid=grid,
        in_specs=[
            pl.BlockSpec((B, BS, D), lambda s: (0, s, 0)),
            pl.BlockSpec((BS, D), lambda s: (s, 0)),
            pl.BlockSpec((1, D), lambda s: (0, 0)),
            pl.BlockSpec((1, D), lambda s: (0, 0)),
        ],
        out_specs=pl.BlockSpec((B, BS, D), lambda s: (0, s, 0)),
        out_shape=jax.ShapeDtypeStruct((B, S, D), x.dtype),
        compiler_params=pltpu.CompilerParams(
            dimension_semantics=("arbitrary",),
        ),
    )(x, pos_table, ln_w.reshape(1, D), ln_b.reshape(1, D))


_B, _S, _D = 4, 8192, 1024
_NC, _NS = 2, 16
_NW = _NC * _NS
_SEG = (_B * _S) // _NW
_T = 16
_NV = _D // 16


def _lane_sum(v):
    lanes = lax.iota(jnp.int32, 16)
    for sh in (8, 4, 2, 1):
        perm = lanes ^ sh
        v = v + lax.gather(
            v, perm[:, None],
            dimension_numbers=lax.GatherDimensionNumbers(
                offset_dims=(), collapsed_slice_dims=(0,),
                start_index_map=(0,)),
            slice_sizes=(1,),
            mode=lax.GatherScatterMode.PROMISE_IN_BOUNDS)
    return v


def _sc_body(x_hbm, pos_hbm, w_hbm, b_hbm, out_hbm,
             xt0, xt1, pt0, pt1, ot0, ot1, wt, bt,
             sem_i0, sem_i1, sem_o0, sem_o1):
    wid = lax.axis_index("s") * _NC + lax.axis_index("c")
    per_b = _NW // _B
    bidx = wid // per_b
    row0 = (wid % per_b) * _SEG
    pltpu.sync_copy(w_hbm, wt)
    pltpu.sync_copy(b_hbm, bt)

    def in_copies(g, xt, pt, sem):
        r0 = row0 + g * _T
        return (pltpu.make_async_copy(x_hbm.at[bidx, pl.ds(r0, _T), :], xt, sem),
                pltpu.make_async_copy(pos_hbm.at[pl.ds(r0, _T), :], pt, sem))

    def out_copy(g, ot, sem):
        r0 = row0 + g * _T
        return pltpu.make_async_copy(ot, out_hbm.at[bidx, pl.ds(r0, _T), :], sem)

    def start_in(g, xt, pt, sem):
        a, b = in_copies(g, xt, pt, sem)
        a.start()
        b.start()

    def wait_in(g, xt, pt, sem):
        a, b = in_copies(g, xt, pt, sem)
        a.wait()
        b.wait()

    def compute(xt, pt, ot):
        for r in range(_T):
            def acc_body(i, carry):
                sv0, qv0, sv1, qv1 = carry
                sl0 = pl.ds(i * 32, 16)
                sl1 = pl.ds(i * 32 + 16, 16)
                v0 = xt[r, sl0] + pt[r, sl0]
                v1 = xt[r, sl1] + pt[r, sl1]
                return sv0 + v0, qv0 + v0 * v0, sv1 + v1, qv1 + v1 * v1

            z = jnp.zeros((16,), jnp.float32)
            sv0, qv0, sv1, qv1 = lax.fori_loop(
                0, _NV // 2, acc_body, (z, z, z, z))
            mean = _lane_sum(sv0 + sv1) * (1.0 / _D)
            var = _lane_sum(qv0 + qv1) * (1.0 / _D) - mean * mean
            xv = var + 1e-5
            bits = lax.bitcast_convert_type(xv, jnp.int32)
            y = lax.bitcast_convert_type(
                jnp.int32(0x5F3759DF) - (bits >> 1), jnp.float32)
            for _i in range(4):
                y = y * (1.5 - 0.5 * xv * y * y)
            scale = y

            def norm_body(i, c):
                sl = pl.ds(i * 16, 16)
                e = xt[r, sl] + pt[r, sl]
                ot[r, sl] = (e - mean) * scale * wt[sl] + bt[sl]
                return c

            lax.fori_loop(0, _NV, norm_body, 0)

    n_pairs = _SEG // _T // 2
    start_in(0, xt0, pt0, sem_i0)
    start_in(1, xt1, pt1, sem_i1)

    def pair_body(h, _):
        g0 = 2 * h
        wait_in(g0, xt0, pt0, sem_i0)

        @pl.when(h > 0)
        def _w0():
            out_copy(g0 - 2, ot0, sem_o0).wait()

        compute(xt0, pt0, ot0)
        out_copy(g0, ot0, sem_o0).start()

        @pl.when(h + 1 < n_pairs)
        def _p0():
            start_in(g0 + 2, xt0, pt0, sem_i0)

        g1 = g0 + 1
        wait_in(g1, xt1, pt1, sem_i1)

        @pl.when(h > 0)
        def _w1():
            out_copy(g1 - 2, ot1, sem_o1).wait()

        compute(xt1, pt1, ot1)
        out_copy(g1, ot1, sem_o1).start()

        @pl.when(h + 1 < n_pairs)
        def _p1():
            start_in(g1 + 2, xt1, pt1, sem_i1)

        return _

    lax.fori_loop(0, n_pairs, pair_body, 0)
    out_copy(2 * n_pairs - 2, ot0, sem_o0).wait()
    out_copy(2 * n_pairs - 1, ot1, sem_o1).wait()


@jax.jit
def _run_sc(x, pos_table, ln_w, ln_b):
    mesh = plsc.VectorSubcoreMesh(core_axis_name="c", subcore_axis_name="s")
    f = functools.partial(
        pl.kernel,
        mesh=mesh,
        out_type=jax.ShapeDtypeStruct((_B, _S, _D), jnp.float32),
        scratch_types=[
            pltpu.VMEM((_T, _D), jnp.float32),
            pltpu.VMEM((_T, _D), jnp.float32),
            pltpu.VMEM((_T, _D), jnp.float32),
            pltpu.VMEM((_T, _D), jnp.float32),
            pltpu.VMEM((_T, _D), jnp.float32),
            pltpu.VMEM((_T, _D), jnp.float32),
            pltpu.VMEM((_D,), jnp.float32),
            pltpu.VMEM((_D,), jnp.float32),
            pltpu.SemaphoreType.DMA,
            pltpu.SemaphoreType.DMA,
            pltpu.SemaphoreType.DMA,
            pltpu.SemaphoreType.DMA,
        ],
    )(_sc_body)
    return f(x, pos_table, ln_w, ln_b)


def kernel(x, batch_size, pos_table, ln_w, ln_b):
    return _run_sc(x, pos_table, ln_w, ln_b)


# speedup vs baseline: 5.9091x; 5.9091x over previous
import functools

import jax
import jax.numpy as jnp
from jax import lax
from jax.experimental import pallas as pl
from jax.experimental.pallas import tpu as pltpu
from jax.experimental.pallas import tpu_sc as plsc

BS = 512


def _ln_kernel(x_ref, p_ref, w_ref, b_ref, o_ref):
    e = x_ref[...] + p_ref[None]
    mean = jnp.mean(e, axis=-1, keepdims=True)
    c = e - mean
    var = jnp.mean(c * c, axis=-1, keepdims=True)
    inv = jax.lax.rsqrt(var + 1e-5)
    o_ref[...] = (c * inv) * w_ref[0] + b_ref[0]


@jax.jit
def _run_tc(x, pos_table, ln_w, ln_b):
    B, S, D = x.shape
    grid = (S // BS,)
    return pl.pallas_call(
        _ln_kernel,
        grid=grid,
        in_specs=[
            pl.BlockSpec((B, BS, D), lambda s: (0, s, 0)),
            pl.BlockSpec((BS, D), lambda s: (s, 0)),
            pl.BlockSpec((1, D), lambda s: (0, 0)),
            pl.BlockSpec((1, D), lambda s: (0, 0)),
        ],
        out_specs=pl.BlockSpec((B, BS, D), lambda s: (0, s, 0)),
        out_shape=jax.ShapeDtypeStruct((B, S, D), x.dtype),
        compiler_params=pltpu.CompilerParams(
            dimension_semantics=("arbitrary",),
        ),
    )(x, pos_table, ln_w.reshape(1, D), ln_b.reshape(1, D))


_B, _S, _D = 4, 8192, 1024
_NC, _NS = 2, 16
_NW = _NC * _NS
_SEG = (_B * _S) // _NW
_T = 16
_NV = _D // 16


def _lane_sum(v):
    lanes = lax.iota(jnp.int32, 16)
    for sh in (8, 4, 2, 1):
        perm = lanes ^ sh
        v = v + lax.gather(
            v, perm[:, None],
            dimension_numbers=lax.GatherDimensionNumbers(
                offset_dims=(), collapsed_slice_dims=(0,),
                start_index_map=(0,)),
            slice_sizes=(1,),
            mode=lax.GatherScatterMode.PROMISE_IN_BOUNDS)
    return v


def _sc_body(x_hbm, pos_hbm, w_hbm, b_hbm, out_hbm,
             xt0, xt1, pt0, pt1, ot0, ot1, wt, bt,
             sem_i0, sem_i1, sem_o0, sem_o1):
    wid = lax.axis_index("s") * _NC + lax.axis_index("c")
    per_b = _NW // _B
    bidx = wid // per_b
    row0 = (wid % per_b) * _SEG
    pltpu.sync_copy(w_hbm, wt)
    pltpu.sync_copy(b_hbm, bt)

    def in_copies(g, xt, pt, sem):
        r0 = row0 + g * _T
        return (pltpu.make_async_copy(x_hbm.at[bidx, pl.ds(r0, _T), :], xt, sem),
                pltpu.make_async_copy(pos_hbm.at[pl.ds(r0, _T), :], pt, sem))

    def out_copy(g, ot, sem):
        r0 = row0 + g * _T
        return pltpu.make_async_copy(ot, out_hbm.at[bidx, pl.ds(r0, _T), :], sem)

    def start_in(g, xt, pt, sem):
        a, b = in_copies(g, xt, pt, sem)
        a.start()
        b.start()

    def wait_in(g, xt, pt, sem):
        a, b = in_copies(g, xt, pt, sem)
        a.wait()
        b.wait()

    def compute(xt, pt, ot):
        for r in range(_T):
            def acc_body(i, carry):
                sv0, qv0, sv1, qv1 = carry
                sl0 = pl.ds(i * 32, 16)
                sl1 = pl.ds(i * 32 + 16, 16)
                v0 = xt[r, sl0] + pt[r, sl0]
                v1 = xt[r, sl1] + pt[r, sl1]
                return sv0 + v0, qv0 + v0 * v0, sv1 + v1, qv1 + v1 * v1

            z = jnp.zeros((16,), jnp.float32)
            sv0, qv0, sv1, qv1 = lax.fori_loop(
                0, _NV // 2, acc_body, (z, z, z, z))
            mean = _lane_sum(sv0 + sv1) * (1.0 / _D)
            var = _lane_sum(qv0 + qv1) * (1.0 / _D) - mean * mean
            xv = var + 1e-5
            bits = lax.bitcast_convert_type(xv, jnp.int32)
            y = lax.bitcast_convert_type(
                jnp.int32(0x5F3759DF) - (bits >> 1), jnp.float32)
            for _i in range(4):
                y = y * (1.5 - 0.5 * xv * y * y)
            scale = y

            def norm_body(i, c):
                sl = pl.ds(i * 16, 16)
                e = xt[r, sl] + pt[r, sl]
                ot[r, sl] = (e - mean) * scale * wt[sl] + bt[sl]
                return c

            lax.fori_loop(0, _NV, norm_body, 0)

    n_pairs = _SEG // _T // 2
    start_in(0, xt0, pt0, sem_i0)
    start_in(1, xt1, pt1, sem_i1)

    def pair_body(h, _):
        g0 = 2 * h
        wait_in(g0, xt0, pt0, sem_i0)

        @pl.when(h > 0)
        def _w0():
            out_copy(g0 - 2, ot0, sem_o0).wait()

        compute(xt0, pt0, ot0)
        out_copy(g0, ot0, sem_o0).start()

        @pl.when(h + 1 < n_pairs)
        def _p0():
            start_in(g0 + 2, xt0, pt0, sem_i0)

        g1 = g0 + 1
        wait_in(g1, xt1, pt1, sem_i1)

        @pl.when(h > 0)
        def _w1():
            out_copy(g1 - 2, ot1, sem_o1).wait()

        compute(xt1, pt1, ot1)
        out_copy(g1, ot1, sem_o1).start()

        @pl.when(h + 1 < n_pairs)
        def _p1():
            start_in(g1 + 2, xt1, pt1, sem_i1)

        return _

    lax.fori_loop(0, n_pairs, pair_body, 0)
    out_copy(2 * n_pairs - 2, ot0, sem_o0).wait()
    out_copy(2 * n_pairs - 1, ot1, sem_o1).wait()


@jax.jit
def _run_sc(x, pos_table, ln_w, ln_b):
    mesh = plsc.VectorSubcoreMesh(core_axis_name="c", subcore_axis_name="s")
    f = functools.partial(
        pl.kernel,
        mesh=mesh,
        out_type=jax.ShapeDtypeStruct((_B, _S, _D), jnp.float32),
        scratch_types=[
            pltpu.VMEM((_T, _D), jnp.float32),
            pltpu.VMEM((_T, _D), jnp.float32),
            pltpu.VMEM((_T, _D), jnp.float32),
            pltpu.VMEM((_T, _D), jnp.float32),
            pltpu.VMEM((_T, _D), jnp.float32),
            pltpu.VMEM((_T, _D), jnp.float32),
            pltpu.VMEM((_D,), jnp.float32),
            pltpu.VMEM((_D,), jnp.float32),
            pltpu.SemaphoreType.DMA,
            pltpu.SemaphoreType.DMA,
            pltpu.SemaphoreType.DMA,
            pltpu.SemaphoreType.DMA,
        ],
    )(_sc_body)
    return f(x, pos_table, ln_w, ln_b)


def kernel(x, batch_size, pos_table, ln_w, ln_b):
    return _run_tc(x, pos_table, ln_w, ln_b)
